# baseline (speedup 1.0000x reference)
import functools

import jax
import jax.numpy as jnp
from jax import lax
from jax.experimental import pallas as pl
from jax.experimental.pallas import tpu as pltpu
from jax.experimental.pallas import tpu_sc as plsc

NC = 2
NS = 16
NW = NC * NS

D = 64
ET = D // 8
CB = 128
NBT = CB // 128
LANES = 16
TRP = CB + 1


@functools.partial(jax.jit, static_argnames=("batch", "hist"))
def _embed_gather(idT, table2, batch, hist):
    b_per_w = batch // NW
    ncb = b_per_w // CB
    n_units = hist * ncb
    bt_total = batch // 128
    mesh = plsc.VectorSubcoreMesh(core_axis_name="c", subcore_axis_name="s")

    @functools.partial(
        pl.kernel,
        out_type=jax.ShapeDtypeStruct((hist, ET * bt_total, 8, 128),
                                      jnp.float32),
        mesh=mesh,
        scratch_types=[
            pltpu.VMEM((hist, b_per_w), jnp.int32),
            pltpu.VMEM((CB,), jnp.int32),
            pltpu.VMEM((CB,), jnp.int32),
            pltpu.VMEM((CB, 2 * D), jnp.float32),
            pltpu.VMEM((CB, 2 * D), jnp.float32),
            pltpu.VMEM((D, TRP), jnp.float32),
            pltpu.VMEM((D, TRP), jnp.float32),
            pltpu.SemaphoreType.DMA,
            pltpu.SemaphoreType.DMA,
            pltpu.SemaphoreType.DMA,
            pltpu.SemaphoreType.DMA,
        ],
        compiler_params=pltpu.CompilerParams(
            use_tc_tiling_on_sc=True, needs_layout_passes=False),
    )
    def body(idT_hbm, table_hbm, out_hbm, idx_v, ih0, ih1, r0, r1, t0, t1,
             sg0, sg1, ss0, ss1):
        idxh = [ih0, ih1]
        rows = [r0, r1]
        trs = [t0, t1]
        sg = [sg0, sg1]
        ss = [ss0, ss1]
        wid = lax.axis_index("s") * NC + lax.axis_index("c")
        b_base = wid * b_per_w
        pltpu.sync_copy(idT_hbm.at[:, pl.ds(b_base, b_per_w)], idx_v)

        j = lax.iota(jnp.int32, LANES)
        E = [j + k * LANES for k in range(D // LANES)]

        def prep(i, p):
            h = i // ncb
            c = i % ncb
            for g in range(CB // LANES):
                v = idx_v[h, pl.ds(c * CB + g * LANES, LANES)]
                idxh[p][pl.ds(g * LANES, LANES)] = v >> 1

        def g_copy(i, p):
            return pltpu.make_async_copy(
                table_hbm.at[idxh[p]], rows[p], sg[p])

        def s_copies(i, p):
            h = i // ncb
            c = i % ncb
            bt0 = (b_base + c * CB) // 128
            return [
                pltpu.make_async_copy(
                    trs[p].at[pl.ds(et * 8, 8), pl.ds(btl * 128, 128)],
                    out_hbm.at[h, et * bt_total + bt0 + btl, :, :],
                    ss[p])
                for et in range(ET) for btl in range(NBT)
            ]

        def transpose(i, p):
            h = i // ncb
            c = i % ncb

            def blk(t):
                b0 = t * LANES
                pv = (idx_v[h, pl.ds(c * CB + b0, LANES)] & 1) * D
                for j2 in range(LANES):
                    b = b0 + j2
                    off = pv[j2]
                    bv = jnp.zeros((LANES,), jnp.int32) + b
                    for k in range(D // LANES):
                        v = rows[p][b, pl.ds(off + k * LANES, LANES)]
                        plsc.store_scatter(trs[p], [E[k], bv], v)
            plsc.parallel_loop(0, CB // LANES, 1, unroll=2)(blk)

        prep(0, 0)
        g_copy(0, 0).start()
        prep(1, 1)
        g_copy(1, 1).start()

        def pair(q, carry):
            for p in range(2):
                i = 2 * q + p
                g_copy(i, p).wait()

                @pl.when(i >= 2)
                def _free():
                    for cp in s_copies(i - 2, p):
                        cp.wait()

                transpose(i, p)
                for cp in s_copies(i, p):
                    cp.start()

                @pl.when(i + 2 < n_units)
                def _next():
                    prep(i + 2, p)
                    g_copy(i + 2, p).start()
            return carry

        lax.fori_loop(0, n_units // 2, pair, 0)
        for cp in s_copies(n_units - 2, 0):
            cp.wait()
        for cp in s_copies(n_units - 1, 1):
            cp.wait()

    return body(idT, table2)


def kernel(input_id, table):
    batch, hist = input_id.shape
    vocab = table.shape[0]
    idT = input_id.T.astype(jnp.int32)
    table2 = table.astype(jnp.float32).reshape(vocab // 2, 2 * D)
    out4 = _embed_gather(idT, table2, batch, hist)
    bt = batch // 128
    return (out4.reshape(hist, ET, bt, 8, 128)
            .transpose(2, 4, 0, 1, 3)
            .reshape(batch, hist, D))


# speedup vs baseline: 1.6842x; 1.6842x over previous
import functools

import jax
import jax.numpy as jnp
from jax import lax
from jax.experimental import pallas as pl
from jax.experimental.pallas import tpu as pltpu
from jax.experimental.pallas import tpu_sc as plsc

NC = 2
NS = 16
NW = NC * NS

D = 64
ET = D // 8
CB = 256
NBT = CB // 128
LANES = 16
TRP = CB + 1


@functools.partial(jax.jit, static_argnames=("batch", "hist"))
def _embed_gather(idT, table, batch, hist):
    b_per_w = batch // NW
    ncb = b_per_w // CB
    n_units = hist * ncb
    bt_total = batch // 128
    mesh = plsc.VectorSubcoreMesh(core_axis_name="c", subcore_axis_name="s")

    @functools.partial(
        pl.kernel,
        out_type=jax.ShapeDtypeStruct((hist, ET * bt_total, 8, 128),
                                      jnp.float32),
        mesh=mesh,
        scratch_types=[
            pltpu.VMEM((hist, b_per_w), jnp.int32),
            pltpu.VMEM((CB, D), jnp.float32),
            pltpu.VMEM((CB, D), jnp.float32),
            pltpu.VMEM((D, TRP), jnp.float32),
            pltpu.VMEM((D, TRP), jnp.float32),
            pltpu.SemaphoreType.DMA,
            pltpu.SemaphoreType.DMA,
            pltpu.SemaphoreType.DMA,
            pltpu.SemaphoreType.DMA,
        ],
        compiler_params=pltpu.CompilerParams(
            use_tc_tiling_on_sc=False, needs_layout_passes=False),
    )
    def body(idT_hbm, table_hbm, out_hbm, idx_v, r0, r1, t0, t1,
             sg0, sg1, ss0, ss1):
        rows = [r0, r1]
        trs = [t0, t1]
        sg = [sg0, sg1]
        ss = [ss0, ss1]
        wid = lax.axis_index("s") * NC + lax.axis_index("c")
        b_base = wid * b_per_w
        pltpu.sync_copy(idT_hbm.at[:, pl.ds(b_base, b_per_w)], idx_v)

        j = lax.iota(jnp.int32, LANES)
        E = [j + k * LANES for k in range(D // LANES)]

        def g_copy(i, p):
            h = i // ncb
            c = i % ncb
            return pltpu.make_async_copy(
                table_hbm.at[idx_v.at[h, pl.ds(c * CB, CB)]], rows[p], sg[p])

        def s_copies(i, p):
            h = i // ncb
            c = i % ncb
            bt0 = (b_base + c * CB) // 128
            return [
                pltpu.make_async_copy(
                    trs[p].at[pl.ds(et * 8, 8), pl.ds(btl * 128, 128)],
                    out_hbm.at[h, et * bt_total + bt0 + btl, :, :],
                    ss[p])
                for et in range(ET) for btl in range(NBT)
            ]

        def transpose(p):
            def blk(t):
                b0 = t * 8
                for j2 in range(8):
                    b = b0 + j2
                    bv = jnp.zeros((LANES,), jnp.int32) + b
                    for k in range(D // LANES):
                        v = rows[p][b, pl.ds(k * LANES, LANES)]
                        plsc.store_scatter(trs[p], [E[k], bv], v)
            plsc.parallel_loop(0, CB // 8, 1, unroll=2)(blk)

        g_copy(0, 0).start()
        g_copy(1, 1).start()

        def pair(q, carry):
            for p in range(2):
                i = 2 * q + p
                g_copy(i, p).wait()

                @pl.when(i >= 2)
                def _free():
                    for cp in s_copies(i - 2, p):
                        cp.wait()

                transpose(p)
                for cp in s_copies(i, p):
                    cp.start()

                @pl.when(i + 2 < n_units)
                def _next():
                    g_copy(i + 2, p).start()
            return carry

        lax.fori_loop(0, n_units // 2, pair, 0)
        for cp in s_copies(n_units - 2, 0):
            cp.wait()
        for cp in s_copies(n_units - 1, 1):
            cp.wait()

    return body(idT, table)


def kernel(input_id, table):
    batch, hist = input_id.shape
    idT = input_id.T.astype(jnp.int32)
    out4 = _embed_gather(idT, table.astype(jnp.float32), batch, hist)
    bt = batch // 128
    return (out4.reshape(hist, ET, bt, 8, 128)
            .transpose(2, 4, 0, 1, 3)
            .reshape(batch, hist, D))


# speedup vs baseline: 1.7018x; 1.0105x over previous
import functools

import jax
import jax.numpy as jnp
from jax import lax
from jax.experimental import pallas as pl
from jax.experimental.pallas import tpu as pltpu
from jax.experimental.pallas import tpu_sc as plsc

NC = 2
NS = 16
NW = NC * NS

D = 64
ET = D // 8
CB = 512
NBT = CB // 128
LANES = 16
TRP = CB + 1


@functools.partial(jax.jit, static_argnames=("batch", "hist"))
def _embed_gather(idT, table, batch, hist):
    b_per_w = batch // NW
    ncb = b_per_w // CB
    n_units = hist * ncb
    bt_total = batch // 128
    mesh = plsc.VectorSubcoreMesh(core_axis_name="c", subcore_axis_name="s")

    @functools.partial(
        pl.kernel,
        out_type=jax.ShapeDtypeStruct((hist, ET * bt_total, 8, 128),
                                      jnp.float32),
        mesh=mesh,
        scratch_types=[
            pltpu.VMEM((hist, b_per_w), jnp.int32),
            pltpu.VMEM((CB, D), jnp.float32),
            pltpu.VMEM((CB, D), jnp.float32),
            pltpu.VMEM((D, TRP), jnp.float32),
            pltpu.SemaphoreType.DMA,
            pltpu.SemaphoreType.DMA,
            pltpu.SemaphoreType.DMA,
        ],
        compiler_params=pltpu.CompilerParams(
            use_tc_tiling_on_sc=False, needs_layout_passes=False),
    )
    def body(idT_hbm, table_hbm, out_hbm, idx_v, r0, r1, t0,
             sg0, sg1, ss0):
        rows = [r0, r1]
        trs = [t0]
        sg = [sg0, sg1]
        ss = [ss0]
        wid = lax.axis_index("s") * NC + lax.axis_index("c")
        b_base = wid * b_per_w
        pltpu.sync_copy(idT_hbm.at[:, pl.ds(b_base, b_per_w)], idx_v)

        j = lax.iota(jnp.int32, LANES)
        E = [j + k * LANES for k in range(D // LANES)]

        def g_copy(i, p):
            h = i // ncb
            c = i % ncb
            return pltpu.make_async_copy(
                table_hbm.at[idx_v.at[h, pl.ds(c * CB, CB)]], rows[p], sg[p])

        def s_copies(i):
            h = i // ncb
            c = i % ncb
            bt0 = (b_base + c * CB) // 128
            return [
                pltpu.make_async_copy(
                    trs[0].at[pl.ds(et * 8, 8), pl.ds(btl * 128, 128)],
                    out_hbm.at[h, et * bt_total + bt0 + btl, :, :],
                    ss[0])
                for et in range(ET) for btl in range(NBT)
            ]

        def transpose(p):
            def blk(t):
                b0 = t * 8
                for j2 in range(8):
                    b = b0 + j2
                    bv = jnp.zeros((LANES,), jnp.int32) + b
                    for k in range(D // LANES):
                        v = rows[p][b, pl.ds(k * LANES, LANES)]
                        plsc.store_scatter(trs[0], [E[k], bv], v)
            plsc.parallel_loop(0, CB // 8, 1, unroll=2)(blk)

        g_copy(0, 0).start()
        g_copy(1, 1).start()

        def pair(q, carry):
            for p in range(2):
                i = 2 * q + p
                g_copy(i, p).wait()

                @pl.when(i >= 1)
                def _free():
                    for cp in s_copies(i - 1):
                        cp.wait()

                transpose(p)
                for cp in s_copies(i):
                    cp.start()

                @pl.when(i + 2 < n_units)
                def _next():
                    g_copy(i + 2, p).start()
            return carry

        lax.fori_loop(0, n_units // 2, pair, 0)
        for cp in s_copies(n_units - 1):
            cp.wait()

    return body(idT, table)


def kernel(input_id, table):
    batch, hist = input_id.shape
    idT = input_id.T.astype(jnp.int32)
    out4 = _embed_gather(idT, table.astype(jnp.float32), batch, hist)
    bt = batch // 128
    return (out4.reshape(hist, ET, bt, 8, 128)
            .transpose(2, 4, 0, 1, 3)
            .reshape(batch, hist, D))
